# Initial kernel scaffold; baseline (speedup 1.0000x reference)
import jax
import jax.numpy as jnp
from jax.experimental import pallas as pl


def kernel(Structure, query, m_items, mod_w, mod_b, conv1_w, conv1_b, conv2_w, conv2_b, pe_w, pe_b, pe_g, pe_beta, exp_w, fin_g, fin_b, up_w, up_b, wf_w2, wf_pre_w, wf_post_w, wf_bn_g, wf_bn_b):
    raise NotImplementedError("write your pallas kernel here")


# speedup vs baseline: 2.2844x; 2.2844x over previous
import jax
import jax.numpy as jnp
from jax.experimental import pallas as pl
from jax.experimental.pallas import tpu as pltpu

_HB1 = 32
_HB2 = 32


def _fuse_kernel(st_ref, q_ref, modwT_ref, modb_ref, wtop_ref, wbot_ref,
                 bsn_ref, peflat_ref, peb_ref, peg_ref, pebeta_ref,
                 expw_ref, smean_ref, gall_ref, kvec_ref, coff_ref,
                 wpre_ref, x_ref):
    C = st_ref.shape[1]
    hb = st_ref.shape[2]
    W = st_ref.shape[3]
    P = 4
    npatch = (hb // P) * (W // P)

    st = jnp.transpose(st_ref[0], (1, 2, 0))
    flat = st.reshape(hb * W, C)
    nrm = jnp.sqrt(jnp.sum(flat * flat, axis=-1, keepdims=True))
    s = flat / jnp.maximum(nrm, 1e-12)

    logits = jnp.dot(s, modwT_ref[...], preferred_element_type=jnp.float32)
    sig = jax.nn.sigmoid(logits + modb_ref[...])

    Sn = (jnp.dot(sig, wtop_ref[...], preferred_element_type=jnp.float32)
          + jnp.dot(s, wbot_ref[...], preferred_element_type=jnp.float32)
          + bsn_ref[...])

    snb = Sn.reshape(hb // P, P, W // P, P, C)
    snp = snb.transpose(0, 2, 1, 3, 4).reshape(npatch, P * P * C)
    f0 = jnp.dot(snp, peflat_ref[...], preferred_element_type=jnp.float32)
    f0 = f0 + peb_ref[...]
    mu = jnp.mean(f0, axis=-1, keepdims=True)
    var = jnp.mean((f0 - mu) * (f0 - mu), axis=-1, keepdims=True)
    f = (f0 - mu) * jax.lax.rsqrt(var + 1e-5) * peg_ref[...] + pebeta_ref[...]

    fe = jnp.dot(f, expw_ref[...], preferred_element_type=jnp.float32)
    mean_c = jnp.dot(fe, smean_ref[...], preferred_element_type=jnp.float32)
    msq_c = jnp.dot(fe * fe, smean_ref[...], preferred_element_type=jnp.float32)
    inv_c = jax.lax.rsqrt(msq_c - mean_c * mean_c + 1e-5)

    v = jnp.dot(f, gall_ref[...], preferred_element_type=jnp.float32)
    vr = v.reshape(npatch, P * P, C)
    m1c = ((vr - mean_c[:, :, None] * kvec_ref[...][None, :, :])
           * inv_c[:, :, None] + coff_ref[...][None, :, :])
    m1 = (m1c.reshape(hb // P, W // P, P, P, C)
          .transpose(0, 2, 1, 3, 4).reshape(hb * W, C))

    q = jnp.transpose(q_ref[0], (1, 2, 0)).reshape(hb * W, C)
    xq = jnp.dot(q, wpre_ref[...], preferred_element_type=jnp.float32)
    x = xq + m1
    x_ref[0] = x.reshape(hb, W, C).astype(x_ref.dtype)


def _conv_kernel(xu_ref, xc_ref, xd_ref, wc_ref, bns_ref, bnb_ref, y_ref):
    hb = xc_ref.shape[1]
    W = xc_ref.shape[2]
    C = xc_ref.shape[3]
    i = pl.program_id(1)
    nb = pl.num_programs(1)

    top = xu_ref[0, hb - 1:hb].astype(jnp.float32) * (i > 0).astype(jnp.float32)
    bot = xd_ref[0, 0:1].astype(jnp.float32) * (i < nb - 1).astype(jnp.float32)
    ext = jnp.concatenate([top, xc_ref[0].astype(jnp.float32), bot], axis=0)

    zcol = jnp.zeros((hb + 2, 1, C), jnp.float32)
    a_m = jnp.concatenate([zcol, ext[:, :W - 1, :]], axis=1)
    a_p = jnp.concatenate([ext[:, 1:, :], zcol], axis=1)

    rows = (hb + 2) * W
    taps = (a_m.reshape(rows, C), ext.reshape(rows, C), a_p.reshape(rows, C))
    ss = []
    for di in range(3):
        acc = jnp.zeros((rows, C), jnp.float32)
        for dj in range(3):
            acc = acc + jnp.dot(taps[dj], wc_ref[di, dj],
                                preferred_element_type=jnp.float32)
        ss.append(acc.reshape(hb + 2, W, C))

    y = ss[0][0:hb] + ss[1][1:hb + 1] + ss[2][2:hb + 2]
    y = jnp.clip(y * bns_ref[...] + bnb_ref[...], 0.0, 6.0)
    y_ref[0] = jnp.transpose(y, (2, 0, 1))


def kernel(Structure, query, m_items, mod_w, mod_b, conv1_w, conv1_b,
           conv2_w, conv2_b, pe_w, pe_b, pe_g, pe_beta, exp_w, fin_g,
           fin_b, up_w, up_b, wf_w2, wf_pre_w, wf_post_w, wf_bn_g, wf_bn_b):
    M, C = m_items.shape
    B, _, H, W = Structure.shape
    P = pe_w.shape[-1]
    DS = exp_w.shape[1] // C
    c = C // DS
    nch = P * P
    f32 = jnp.float32

    ww = jax.nn.relu(wf_w2)
    fwt = ww / (ww.sum() + 1e-8)

    mod_wT = mod_w.T
    c1 = conv1_w.reshape(C // 2, M, C)
    w_eff = (c1 * m_items[None]).transpose(1, 2, 0).reshape(M * C, C // 2)
    w_top = jnp.concatenate([w_eff, jnp.zeros((M * C, C // 2), f32)], axis=1)
    w_bot = jnp.concatenate([jnp.zeros((C, C // 2), f32), conv2_w.T], axis=1)
    b_sn = jnp.concatenate([conv1_b, conv2_b]).reshape(1, C)

    pe_flat = pe_w.transpose(2, 3, 1, 0).reshape(P * P * C, C)

    wp = fwt[1] * (fin_g[:, None] * up_w.T)
    kvec = wp.sum(axis=0).reshape(1, C)
    c_off = (fwt[1] * (fin_b @ up_w.T + up_b)).reshape(1, C)
    g_all = jnp.einsum('cjk,ko->cjo', exp_w.reshape(C, nch, c),
                       wp).reshape(C, nch * C)
    s_mean = jnp.repeat(jnp.eye(nch, dtype=f32), c, axis=0) / c
    wf_pre_s = fwt[0] * wf_pre_w.T

    wc = wf_post_w.transpose(2, 3, 1, 0)
    bn_scale = (wf_bn_g / jnp.sqrt(1.0 + 1e-5)).reshape(1, 1, C)
    bn_bias = wf_bn_b.reshape(1, 1, C)

    nb1 = H // _HB1
    full = lambda shape: pl.BlockSpec(shape, lambda b, i: (0,) * len(shape))
    x = pl.pallas_call(
        _fuse_kernel,
        grid=(B, nb1),
        in_specs=[
            pl.BlockSpec((1, C, _HB1, W), lambda b, i: (b, 0, i, 0)),
            pl.BlockSpec((1, C, _HB1, W), lambda b, i: (b, 0, i, 0)),
            full((C, M * C)), full((1, M * C)), full((M * C, C)),
            full((C, C)), full((1, C)), full((P * P * C, C)), full((1, C)),
            full((1, C)), full((1, C)), full((C, DS * C)),
            full((DS * C, nch)), full((C, nch * C)), full((1, C)),
            full((1, C)), full((C, C)),
        ],
        out_specs=pl.BlockSpec((1, _HB1, W, C), lambda b, i: (b, i, 0, 0)),
        out_shape=jax.ShapeDtypeStruct((B, H, W, C), jnp.bfloat16),
        compiler_params=pltpu.CompilerParams(
            dimension_semantics=("parallel", "arbitrary"),
            vmem_limit_bytes=56 * 1024 * 1024,
        ),
    )(Structure, query, mod_wT, mod_b.reshape(1, M * C), w_top, w_bot, b_sn,
      pe_flat, pe_b.reshape(1, C), pe_g.reshape(1, C), pe_beta.reshape(1, C),
      exp_w, s_mean, g_all, kvec, c_off, wf_pre_s)

    nb2 = H // _HB2
    xspec = lambda off: pl.BlockSpec(
        (1, _HB2, W, C),
        lambda b, i: (b, jnp.clip(i + off, 0, nb2 - 1), 0, 0))
    y = pl.pallas_call(
        _conv_kernel,
        grid=(B, nb2),
        in_specs=[
            xspec(-1), xspec(0), xspec(1),
            pl.BlockSpec((3, 3, C, C), lambda b, i: (0, 0, 0, 0)),
            pl.BlockSpec((1, 1, C), lambda b, i: (0, 0, 0)),
            pl.BlockSpec((1, 1, C), lambda b, i: (0, 0, 0)),
        ],
        out_specs=pl.BlockSpec((1, C, _HB2, W), lambda b, i: (b, 0, i, 0)),
        out_shape=jax.ShapeDtypeStruct((B, C, H, W), f32),
        compiler_params=pltpu.CompilerParams(
            dimension_semantics=("parallel", "arbitrary"),
            vmem_limit_bytes=56 * 1024 * 1024,
        ),
    )(x, x, x, wc, bn_scale, bn_bias)
    return y


# speedup vs baseline: 3.8065x; 1.6663x over previous
import jax
import jax.numpy as jnp
from jax.experimental import pallas as pl
from jax.experimental.pallas import tpu as pltpu

_HB1 = 32
_HB2 = 32


def _fuse_kernel(st_ref, q_ref, modwT_ref, modb_ref, wtop_ref, wbot_ref,
                 bsn_ref, peflat_ref, peb_ref, peg_ref, pebeta_ref,
                 expw_ref, smean_ref, gall_ref, kvec_ref, coff_ref,
                 wpre_ref, x_ref):
    C = st_ref.shape[1]
    hb = st_ref.shape[2]
    W = st_ref.shape[3]
    P = 4
    npatch = (hb // P) * (W // P)

    tdot = lambda a, b: jax.lax.dot_general(
        a, b, (((0,), (0,)), ((), ())), preferred_element_type=jnp.float32)

    stm = st_ref[0].reshape(C, hb * W)
    nrm = jnp.sqrt(jnp.sum(stm * stm, axis=0, keepdims=True))
    s_chw = stm / jnp.maximum(nrm, 1e-12)

    logits = tdot(s_chw, modwT_ref[...])
    sig = jax.nn.sigmoid(logits + modb_ref[...])

    Sn = (jnp.dot(sig, wtop_ref[...], preferred_element_type=jnp.float32)
          + tdot(s_chw, wbot_ref[...])
          + bsn_ref[...])

    snb = Sn.reshape(hb // P, P, W // P, P, C)
    snp = snb.transpose(0, 2, 1, 3, 4).reshape(npatch, P * P * C)
    f0 = jnp.dot(snp, peflat_ref[...], preferred_element_type=jnp.float32)
    f0 = f0 + peb_ref[...]
    mu = jnp.mean(f0, axis=-1, keepdims=True)
    var = jnp.mean((f0 - mu) * (f0 - mu), axis=-1, keepdims=True)
    f = (f0 - mu) * jax.lax.rsqrt(var + 1e-5) * peg_ref[...] + pebeta_ref[...]

    fe = jnp.dot(f, expw_ref[...], preferred_element_type=jnp.float32)
    mean_c = jnp.dot(fe, smean_ref[...], preferred_element_type=jnp.float32)
    msq_c = jnp.dot(fe * fe, smean_ref[...], preferred_element_type=jnp.float32)
    inv_c = jax.lax.rsqrt(msq_c - mean_c * mean_c + 1e-5)

    v = jnp.dot(f, gall_ref[...], preferred_element_type=jnp.float32)
    vr = v.reshape(npatch, P * P, C)
    m1c = ((vr - mean_c[:, :, None] * kvec_ref[...][None, :, :])
           * inv_c[:, :, None] + coff_ref[...][None, :, :])
    m1 = (m1c.reshape(hb // P, W // P, P, P, C)
          .transpose(0, 2, 1, 3, 4).reshape(hb * W, C))

    q_chw = q_ref[0].reshape(C, hb * W)
    xq = tdot(q_chw, wpre_ref[...])
    x = xq + m1
    x_ref[0] = x.reshape(hb, W, C).astype(x_ref.dtype)


def _conv_kernel(xu_ref, xc_ref, xd_ref, wc_ref, bns_ref, bnb_ref, y_ref):
    hb = xc_ref.shape[1]
    W = xc_ref.shape[2]
    C = xc_ref.shape[3]
    i = pl.program_id(1)
    nb = pl.num_programs(1)

    top = xu_ref[0, hb - 1:hb].astype(jnp.float32) * (i > 0).astype(jnp.float32)
    bot = xd_ref[0, 0:1].astype(jnp.float32) * (i < nb - 1).astype(jnp.float32)
    ext = jnp.concatenate([top, xc_ref[0].astype(jnp.float32), bot], axis=0)

    zcol = jnp.zeros((hb + 2, 1, C), jnp.float32)
    a_m = jnp.concatenate([zcol, ext[:, :W - 1, :]], axis=1)
    a_p = jnp.concatenate([ext[:, 1:, :], zcol], axis=1)

    rows = (hb + 2) * W
    taps = (a_m.reshape(rows, C), ext.reshape(rows, C), a_p.reshape(rows, C))
    ss = []
    for di in range(3):
        acc = jnp.zeros((rows, C), jnp.float32)
        for dj in range(3):
            acc = acc + jnp.dot(taps[dj], wc_ref[di, dj],
                                preferred_element_type=jnp.float32)
        ss.append(acc.reshape(hb + 2, W, C))

    y = ss[0][0:hb] + ss[1][1:hb + 1] + ss[2][2:hb + 2]
    y = jnp.clip(y * bns_ref[...] + bnb_ref[...], 0.0, 6.0)
    y_ref[0] = jnp.transpose(y, (2, 0, 1))


def kernel(Structure, query, m_items, mod_w, mod_b, conv1_w, conv1_b,
           conv2_w, conv2_b, pe_w, pe_b, pe_g, pe_beta, exp_w, fin_g,
           fin_b, up_w, up_b, wf_w2, wf_pre_w, wf_post_w, wf_bn_g, wf_bn_b):
    M, C = m_items.shape
    B, _, H, W = Structure.shape
    P = pe_w.shape[-1]
    DS = exp_w.shape[1] // C
    c = C // DS
    nch = P * P
    f32 = jnp.float32

    ww = jax.nn.relu(wf_w2)
    fwt = ww / (ww.sum() + 1e-8)

    mod_wT = mod_w.T
    c1 = conv1_w.reshape(C // 2, M, C)
    w_eff = (c1 * m_items[None]).transpose(1, 2, 0).reshape(M * C, C // 2)
    w_top = jnp.concatenate([w_eff, jnp.zeros((M * C, C // 2), f32)], axis=1)
    w_bot = jnp.concatenate([jnp.zeros((C, C // 2), f32), conv2_w.T], axis=1)
    b_sn = jnp.concatenate([conv1_b, conv2_b]).reshape(1, C)

    pe_flat = pe_w.transpose(2, 3, 1, 0).reshape(P * P * C, C)

    wp = fwt[1] * (fin_g[:, None] * up_w.T)
    kvec = wp.sum(axis=0).reshape(1, C)
    c_off = (fwt[1] * (fin_b @ up_w.T + up_b)).reshape(1, C)
    g_all = jnp.einsum('cjk,ko->cjo', exp_w.reshape(C, nch, c),
                       wp).reshape(C, nch * C)
    s_mean = jnp.repeat(jnp.eye(nch, dtype=f32), c, axis=0) / c
    wf_pre_s = fwt[0] * wf_pre_w.T

    wc = wf_post_w.transpose(2, 3, 1, 0)
    bn_scale = (wf_bn_g / jnp.sqrt(1.0 + 1e-5)).reshape(1, 1, C)
    bn_bias = wf_bn_b.reshape(1, 1, C)

    nb1 = H // _HB1
    full = lambda shape: pl.BlockSpec(shape, lambda b, i: (0,) * len(shape))
    x = pl.pallas_call(
        _fuse_kernel,
        grid=(B, nb1),
        in_specs=[
            pl.BlockSpec((1, C, _HB1, W), lambda b, i: (b, 0, i, 0)),
            pl.BlockSpec((1, C, _HB1, W), lambda b, i: (b, 0, i, 0)),
            full((C, M * C)), full((1, M * C)), full((M * C, C)),
            full((C, C)), full((1, C)), full((P * P * C, C)), full((1, C)),
            full((1, C)), full((1, C)), full((C, DS * C)),
            full((DS * C, nch)), full((C, nch * C)), full((1, C)),
            full((1, C)), full((C, C)),
        ],
        out_specs=pl.BlockSpec((1, _HB1, W, C), lambda b, i: (b, i, 0, 0)),
        out_shape=jax.ShapeDtypeStruct((B, H, W, C), jnp.bfloat16),
        compiler_params=pltpu.CompilerParams(
            dimension_semantics=("parallel", "arbitrary"),
            vmem_limit_bytes=56 * 1024 * 1024,
        ),
    )(Structure, query, mod_wT, mod_b.reshape(1, M * C), w_top, w_bot, b_sn,
      pe_flat, pe_b.reshape(1, C), pe_g.reshape(1, C), pe_beta.reshape(1, C),
      exp_w, s_mean, g_all, kvec, c_off, wf_pre_s)

    nb2 = H // _HB2
    xspec = lambda off: pl.BlockSpec(
        (1, _HB2, W, C),
        lambda b, i: (b, jnp.clip(i + off, 0, nb2 - 1), 0, 0))
    y = pl.pallas_call(
        _conv_kernel,
        grid=(B, nb2),
        in_specs=[
            xspec(-1), xspec(0), xspec(1),
            pl.BlockSpec((3, 3, C, C), lambda b, i: (0, 0, 0, 0)),
            pl.BlockSpec((1, 1, C), lambda b, i: (0, 0, 0)),
            pl.BlockSpec((1, 1, C), lambda b, i: (0, 0, 0)),
        ],
        out_specs=pl.BlockSpec((1, C, _HB2, W), lambda b, i: (b, 0, i, 0)),
        out_shape=jax.ShapeDtypeStruct((B, C, H, W), f32),
        compiler_params=pltpu.CompilerParams(
            dimension_semantics=("parallel", "arbitrary"),
            vmem_limit_bytes=56 * 1024 * 1024,
        ),
    )(x, x, x, wc, bn_scale, bn_bias)
    return y
